# baseline (speedup 1.0000x reference)
import functools

import jax
import jax.numpy as jnp
from jax import lax
from jax.experimental import pallas as pl
from jax.experimental.pallas import tpu as pltpu
from jax.experimental.pallas import tpu_sc as plsc

_NB = 64
_NH = 4
_S = 2048
_D = 64
_NPROB = 32
_NITEM = _NH * _S
_NG = _NH * _NB
_CW = 32
_QB = 256
_KB = _QB + 64
_PAD = 33 * _QB
_CHUNK = 128
_LANES = 16
_W = 2 * _D


def _hash_body(q_ref, rot_ref, g_ref, hist_ref):
    q = q_ref[0]
    rot = rot_ref[...]
    r = lax.dot_general(q, rot, (((1,), (0,)), ((), ())),
                        preferred_element_type=jnp.float32)
    for h in range(_NH):
        rh = r[:, h * (_NB // 2):(h + 1) * (_NB // 2)]
        both = jnp.concatenate([rh, -rh], axis=1)
        bh = jnp.argmax(both, axis=1).astype(jnp.int32)
        g_ref[0, h, :] = bh + h * _NB
        onehot = (bh[:, None] ==
                  lax.broadcasted_iota(jnp.int32, (_S, _NB), 1))
        hist_ref[0, 0, pl.ds(h * _NB, _NB)] = jnp.sum(
            onehot.astype(jnp.int32), axis=0)


def _hash_tc(q_p, rot2d):
    return pl.pallas_call(
        _hash_body,
        grid=(_NPROB,),
        in_specs=[
            pl.BlockSpec((1, _S, _D), lambda p: (p, 0, 0)),
            pl.BlockSpec((_D, _NH * _NB // 2), lambda p: (0, 0)),
        ],
        out_specs=[
            pl.BlockSpec((1, _NH, _S), lambda p: (p, 0, 0)),
            pl.BlockSpec((1, 1, _NG), lambda p: (p, 0, 0)),
        ],
        out_shape=[
            jax.ShapeDtypeStruct((_NPROB, _NH, _S), jnp.int32),
            jax.ShapeDtypeStruct((_NPROB, 1, _NG), jnp.int32),
        ],
    )(q_p, rot2d)


def _sort_gather_body(g_hbm, hist_hbm, qv_hbm,
                      sqv_hbm, sqvu_hbm, nd_hbm, ipos_hbm,
                      g_v, hist_v, start_v, ipos_v, st_v, nd_v, idx_v,
                      rows_v, sem):
    p = lax.axis_index("s") * 2 + lax.axis_index("c")
    pltpu.sync_copy(g_hbm.at[pl.ds(p * _NITEM, _NITEM)], g_v)
    pltpu.sync_copy(hist_hbm.at[pl.ds(p * _NG, _NG)], hist_v)

    carry = jnp.int32(0)
    for bi in range(_NG // _LANES):
        h16 = hist_v[pl.ds(bi * _LANES, _LANES)]
        inc = plsc.cumsum(h16)
        start_v[pl.ds(bi * _LANES, _LANES)] = inc - h16 + carry
        carry = carry + inc[_LANES - 1]

    lane = lax.iota(jnp.int32, _LANES)

    def sort_step(i, _):
        g16 = g_v[pl.ds(i * _LANES, _LANES)]
        item = i * _LANES + lane
        skey, sitem = plsc.sort_key_val(g16 * _LANES + lane, item)
        gs = skey >> 4
        gprev = gs.at[jnp.maximum(lane - 1, 0)].get(mode="promise_in_bounds")
        run_start = (gs != gprev) | (lane == 0)
        rank = lane - plsc.cummax(jnp.where(run_start, lane, 0))
        c = plsc.load_gather(start_v, [gs])
        pos = c + rank
        plsc.store_scatter(ipos_v, [sitem], pos)
        plsc.store_scatter(st_v, [pos], sitem & (_S - 1))
        gnext = gs.at[jnp.minimum(lane + 1, _LANES - 1)].get(
            mode="promise_in_bounds")
        last = (gs != gnext) | (lane == _LANES - 1)
        plsc.store_scatter(start_v, [gs], pos + 1, mask=last)
        return 0

    lax.fori_loop(0, _NITEM // _LANES, sort_step, 0)

    pltpu.sync_copy(ipos_v, ipos_hbm.at[pl.ds(p * _NITEM, _NITEM)])

    def nd_step(i, _):
        s16 = i * _LANES + lane
        t16 = st_v[pl.ds(i * _LANES, _LANES)]
        c16 = s16 >> 5
        acc = jnp.zeros((_LANES,), jnp.int32)
        for hh in range(_NH):
            posh = plsc.load_gather(ipos_v, [t16 + hh * _S])
            dc = ((posh >> 5) - c16) & 255
            acc = acc + jnp.where((dc <= 1) | (dc == 255),
                                  jnp.int32(1), jnp.int32(0))
        nd_v[pl.ds(i * _LANES, _LANES)] = acc
        return 0

    lax.fori_loop(0, _NITEM // _LANES, nd_step, 0)
    pltpu.sync_copy(nd_v, nd_hbm.at[pl.ds(p * _NITEM, _NITEM)])

    def gather_rows(src_off, n, dst_off, unpadded):
        def mk_idx(j, _):
            st16 = st_v[pl.ds(src_off + j * _LANES, _LANES)]
            idx_v[pl.ds(j * _LANES, _LANES)] = p * _S + st16
            return 0
        lax.fori_loop(0, n // _LANES, mk_idx, 0)
        pltpu.async_copy(qv_hbm.at[idx_v.at[pl.ds(0, n)]],
                         rows_v.at[pl.ds(0, n)], sem).wait()
        pltpu.sync_copy(rows_v.at[pl.ds(0, n)],
                        sqv_hbm.at[pl.ds(p * _PAD + dst_off, n)])
        if unpadded:
            pltpu.sync_copy(rows_v.at[pl.ds(0, n)],
                            sqvu_hbm.at[pl.ds(p * _NITEM + src_off, n)])

    def chunk_step(c, _):
        gather_rows(c * _CHUNK, _CHUNK, 32 + c * _CHUNK, True)
        return 0

    lax.fori_loop(0, _NITEM // _CHUNK, chunk_step, 0)
    gather_rows(_NITEM - 32, 32, 0, False)
    gather_rows(0, 32, _NITEM + 32, False)


def _sort_gather_sc(g_flat, hist_flat, qv_tab):
    mesh = plsc.VectorSubcoreMesh(core_axis_name="c", subcore_axis_name="s")
    f = pl.kernel(
        _sort_gather_body,
        mesh=mesh,
        compiler_params=pltpu.CompilerParams(needs_layout_passes=False),
        out_type=[
            jax.ShapeDtypeStruct((_NPROB * _PAD, _W), jnp.float32),
            jax.ShapeDtypeStruct((_NPROB * _NITEM, _W), jnp.float32),
            jax.ShapeDtypeStruct((_NPROB * _NITEM,), jnp.int32),
            jax.ShapeDtypeStruct((_NPROB * _NITEM,), jnp.int32),
        ],
        scratch_types=[
            pltpu.VMEM((_NITEM,), jnp.int32),
            pltpu.VMEM((_NG,), jnp.int32),
            pltpu.VMEM((_NG,), jnp.int32),
            pltpu.VMEM((_NITEM,), jnp.int32),
            pltpu.VMEM((_NITEM,), jnp.int32),
            pltpu.VMEM((_NITEM,), jnp.int32),
            pltpu.VMEM((_CHUNK,), jnp.int32),
            pltpu.VMEM((_CHUNK, _W), jnp.float32),
            pltpu.SemaphoreType.DMA,
        ],
    )
    return f(g_flat, hist_flat, qv_tab)


_L2E = 1.4426950408889634
_SB = 64


def _attn_body(m64_ref, q_ref, a_ref, b_ref, nd_ref, x_ref):
    col = lax.broadcasted_iota(jnp.int32, (_SB, _W), 1)
    qhalf = col >= _D
    mask = m64_ref[...]
    for s in range(_QB // _SB):
        qrow = q_ref[0, pl.ds(s * _SB, _SB), :]
        qz = jnp.where(qhalf, qrow * (_L2E / (_D ** 0.5)), 0.0)
        m2 = jnp.sum(qz * qrow, axis=1, keepdims=True)
        nd = (nd_ref[0, 0, 0, pl.ds(s * _SB, _SB)]
              .astype(jnp.float32)[:, None] - 1.0)
        if s < 3:
            w = a_ref[0, pl.ds(s * _SB, 2 * _SB), :]
            dots = lax.dot_general(qz, w, (((1,), (1,)), ((), ())),
                                   preferred_element_type=jnp.float32)
            e = jnp.exp2(dots - m2) * mask
            ssum = jnp.sum(e, axis=1, keepdims=True) - nd
            x_full = lax.dot_general(e, w, (((1,), (0,)), ((), ())),
                                     preferred_element_type=jnp.float32)
        else:
            wa = a_ref[0, pl.ds(3 * _SB, _SB), :]
            wb = b_ref[0]
            d1 = lax.dot_general(qz, wa, (((1,), (1,)), ((), ())),
                                 preferred_element_type=jnp.float32)
            d2 = lax.dot_general(qz, wb, (((1,), (1,)), ((), ())),
                                 preferred_element_type=jnp.float32)
            e1 = jnp.exp2(d1 - m2) * mask[:, :_SB]
            e2 = jnp.exp2(d2 - m2) * mask[:, _SB:]
            ssum = (jnp.sum(e1, axis=1, keepdims=True)
                    + jnp.sum(e2, axis=1, keepdims=True)) - nd
            x_full = (lax.dot_general(e1, wa, (((1,), (0,)), ((), ())),
                                      preferred_element_type=jnp.float32)
                      + lax.dot_general(e2, wb, (((1,), (0,)), ((), ())),
                                        preferred_element_type=jnp.float32))
        x_full = x_full - nd * qrow
        out = jnp.where(qhalf, jnp.log(ssum) + m2 * (1.0 / _L2E),
                        x_full / ssum)
        x_ref[0, pl.ds(s * _SB, _SB), :] = out


def _attn_tc(sqv_pad, sqv_unpad, ndup):
    nd4 = ndup.reshape(_NPROB, _NITEM // _QB, 1, _QB)
    nj = _NITEM // _QB
    r = lax.broadcasted_iota(jnp.int32, (_SB, 2 * _SB), 0)
    ccol = lax.broadcasted_iota(jnp.int32, (_SB, 2 * _SB), 1)
    lo = (r // _CW) * _CW
    m64 = jnp.where((ccol >= lo) & (ccol < lo + 3 * _CW) & (ccol != r + 32),
                    1.0, 0.0).astype(jnp.float32)
    return pl.pallas_call(
        _attn_body,
        grid=(_NPROB, nj),
        in_specs=[
            pl.BlockSpec((_SB, 2 * _SB), lambda p, j: (0, 0)),
            pl.BlockSpec((1, _QB, _W), lambda p, j: (p, j, 0)),
            pl.BlockSpec((1, _QB, _W), lambda p, j: (p, j, 0)),
            pl.BlockSpec((1, 64, _W), lambda p, j: (p, 4 * j + 4, 0)),
            pl.BlockSpec((1, 1, 1, _QB), lambda p, j: (p, j, 0, 0)),
        ],
        out_specs=pl.BlockSpec((1, _QB, _W), lambda p, j: (p, j, 0)),
        out_shape=jax.ShapeDtypeStruct((_NPROB, _NITEM, _W), jnp.float32),
    )(m64, sqv_unpad.reshape(_NPROB, _NITEM, _W), sqv_pad, sqv_pad, nd4)


def _unsort_body(xtab_hbm, ipos_hbm, o_hbm, ipos_v, idx_v, rows_v, sem):
    p = lax.axis_index("s") * 2 + lax.axis_index("c")
    pltpu.sync_copy(ipos_hbm.at[pl.ds(p * _NITEM, _NITEM)], ipos_v)

    def chunk_step(c, _):
        base = c * _CHUNK

        def mk_idx(j, _):
            pos16 = ipos_v[pl.ds(base + j * _LANES, _LANES)]
            idx_v[pl.ds(j * _LANES, _LANES)] = p * _NITEM + pos16
            return 0

        lax.fori_loop(0, _CHUNK // _LANES, mk_idx, 0)
        pltpu.async_copy(xtab_hbm.at[idx_v], rows_v, sem).wait()
        pltpu.sync_copy(rows_v, o_hbm.at[pl.ds(p * _NITEM + base, _CHUNK)])
        return 0

    lax.fori_loop(0, _NITEM // _CHUNK, chunk_step, 0)


def _unsort_sc(xtab, ipos):
    mesh = plsc.VectorSubcoreMesh(core_axis_name="c", subcore_axis_name="s")
    f = pl.kernel(
        _unsort_body,
        mesh=mesh,
        compiler_params=pltpu.CompilerParams(needs_layout_passes=False),
        out_type=jax.ShapeDtypeStruct((_NPROB * _NITEM, _W), jnp.float32),
        scratch_types=[
            pltpu.VMEM((_NITEM,), jnp.int32),
            pltpu.VMEM((_CHUNK,), jnp.int32),
            pltpu.VMEM((_CHUNK, _W), jnp.float32),
            pltpu.SemaphoreType.DMA,
        ],
    )
    return f(xtab, ipos)


def _combine_body(o_ref, out_ref):
    blk = o_ref[0]
    o = blk[:, :, :_D]
    lgb = blk[:, :, _D:]
    m = jnp.max(lgb, axis=0, keepdims=True)
    e = jnp.exp(lgb - m)
    w = e / jnp.sum(e, axis=0, keepdims=True)
    out_ref[0] = jnp.sum(w * o, axis=0)


def _combine_tc(o_ext):
    o4 = o_ext.reshape(_NPROB, _NH, _S, _W)
    return pl.pallas_call(
        _combine_body,
        grid=(_NPROB, _S // 1024),
        in_specs=[
            pl.BlockSpec((1, _NH, 1024, _W), lambda p, t: (p, 0, t, 0)),
        ],
        out_specs=pl.BlockSpec((1, 1024, _D), lambda p, t: (p, t, 0)),
        out_shape=jax.ShapeDtypeStruct((_NPROB, _S, _D), jnp.float32),
    )(o4)


def kernel(query, key, value, rotations):
    del key
    B, S, H, D = query.shape
    q_p = query.transpose(0, 2, 1, 3).reshape(_NPROB, _S, _D)
    v_p = value.transpose(0, 2, 1, 3).reshape(_NPROB, _S, _D)
    qv_tab = jnp.concatenate([v_p, q_p], axis=-1).reshape(_NPROB * _S, _W)
    rot2d = rotations.reshape(_D, _NH * (_NB // 2))

    g, hist = _hash_tc(q_p, rot2d)
    sqv_flat, sqvu_flat, ndup, ipos = _sort_gather_sc(
        g.reshape(-1), hist.reshape(-1), qv_tab)
    x_ext = _attn_tc(sqv_flat.reshape(_NPROB, _PAD, _W), sqvu_flat, ndup)
    o_ext = _unsort_sc(x_ext.reshape(_NPROB * _NITEM, _W), ipos)
    out_p = _combine_tc(o_ext)
    return out_p.reshape(B, H, S, D).transpose(0, 2, 1, 3)


# speedup vs baseline: 1.1491x; 1.1491x over previous
import functools

import jax
import jax.numpy as jnp
from jax import lax
from jax.experimental import pallas as pl
from jax.experimental.pallas import tpu as pltpu
from jax.experimental.pallas import tpu_sc as plsc

_NB = 64
_NH = 4
_S = 2048
_D = 64
_NPROB = 32
_NITEM = _NH * _S
_NG = _NH * _NB
_CW = 32
_QB = 256
_KB = _QB + 64
_PAD = 33 * _QB
_CHUNK = 128
_LANES = 16
_W = 2 * _D


def _hash_body(q_ref, rot_ref, g_ref, hist_ref):
    q = q_ref[0]
    rot = rot_ref[...]
    r = lax.dot_general(q, rot, (((1,), (0,)), ((), ())),
                        preferred_element_type=jnp.float32)
    for h in range(_NH):
        rh = r[:, h * (_NB // 2):(h + 1) * (_NB // 2)]
        both = jnp.concatenate([rh, -rh], axis=1)
        bh = jnp.argmax(both, axis=1).astype(jnp.int32)
        g_ref[0, h, :] = bh + h * _NB
        onehot = (bh[:, None] ==
                  lax.broadcasted_iota(jnp.int32, (_S, _NB), 1))
        hist_ref[0, 0, pl.ds(h * _NB, _NB)] = jnp.sum(
            onehot.astype(jnp.int32), axis=0)


def _hash_tc(q_p, rot2d):
    return pl.pallas_call(
        _hash_body,
        grid=(_NPROB,),
        in_specs=[
            pl.BlockSpec((1, _S, _D), lambda p: (p, 0, 0)),
            pl.BlockSpec((_D, _NH * _NB // 2), lambda p: (0, 0)),
        ],
        out_specs=[
            pl.BlockSpec((1, _NH, _S), lambda p: (p, 0, 0)),
            pl.BlockSpec((1, 1, _NG), lambda p: (p, 0, 0)),
        ],
        out_shape=[
            jax.ShapeDtypeStruct((_NPROB, _NH, _S), jnp.int32),
            jax.ShapeDtypeStruct((_NPROB, 1, _NG), jnp.int32),
        ],
    )(q_p, rot2d)


def _sort_gather_body(g_hbm, hist_hbm, qv_hbm,
                      sqv_hbm, tok_hbm, ipos_hbm,
                      g_v, hist_v, start_v, ipos_v, st_v, idx_v,
                      rows_v, sem):
    p = lax.axis_index("s") * 2 + lax.axis_index("c")
    pltpu.sync_copy(g_hbm.at[pl.ds(p * _NITEM, _NITEM)], g_v)
    pltpu.sync_copy(hist_hbm.at[pl.ds(p * _NG, _NG)], hist_v)

    carry = jnp.int32(0)
    for bi in range(_NG // _LANES):
        h16 = hist_v[pl.ds(bi * _LANES, _LANES)]
        inc = plsc.cumsum(h16)
        start_v[pl.ds(bi * _LANES, _LANES)] = inc - h16 + carry
        carry = carry + inc[_LANES - 1]

    lane = lax.iota(jnp.int32, _LANES)

    def sort_step(i, _):
        g16 = g_v[pl.ds(i * _LANES, _LANES)]
        item = i * _LANES + lane
        skey, sitem = plsc.sort_key_val(g16 * _LANES + lane, item)
        gs = skey >> 4
        gprev = gs.at[jnp.maximum(lane - 1, 0)].get(mode="promise_in_bounds")
        run_start = (gs != gprev) | (lane == 0)
        rank = lane - plsc.cummax(jnp.where(run_start, lane, 0))
        c = plsc.load_gather(start_v, [gs])
        pos = c + rank
        plsc.store_scatter(ipos_v, [sitem], pos)
        plsc.store_scatter(st_v, [pos], sitem & (_S - 1))
        gnext = gs.at[jnp.minimum(lane + 1, _LANES - 1)].get(
            mode="promise_in_bounds")
        last = (gs != gnext) | (lane == _LANES - 1)
        plsc.store_scatter(start_v, [gs], pos + 1, mask=last)
        return 0

    lax.fori_loop(0, _NITEM // _LANES, sort_step, 0)

    pltpu.sync_copy(ipos_v, ipos_hbm.at[pl.ds(p * _NITEM, _NITEM)])

    def gather_rows(src_off, n, dst_off):
        def mk_idx(j, _):
            st16 = st_v[pl.ds(src_off + j * _LANES, _LANES)]
            idx_v[pl.ds(j * _LANES, _LANES)] = p * _S + st16
            return 0
        lax.fori_loop(0, n // _LANES, mk_idx, 0)
        pltpu.async_copy(qv_hbm.at[idx_v.at[pl.ds(0, n)]],
                         rows_v.at[pl.ds(0, n)], sem).wait()
        pltpu.sync_copy(rows_v.at[pl.ds(0, n)],
                        sqv_hbm.at[pl.ds(p * _PAD + dst_off, n)])
        pltpu.sync_copy(st_v.at[pl.ds(src_off, n)],
                        tok_hbm.at[pl.ds(p * _PAD + dst_off, n)])

    def chunk_step(c, _):
        gather_rows(c * _CHUNK, _CHUNK, 32 + c * _CHUNK)
        return 0

    lax.fori_loop(0, _NITEM // _CHUNK, chunk_step, 0)
    gather_rows(_NITEM - 32, 32, 0)
    gather_rows(0, 32, _NITEM + 32)


def _sort_gather_sc(g_flat, hist_flat, qv_tab):
    mesh = plsc.VectorSubcoreMesh(core_axis_name="c", subcore_axis_name="s")
    f = pl.kernel(
        _sort_gather_body,
        mesh=mesh,
        compiler_params=pltpu.CompilerParams(needs_layout_passes=False),
        out_type=[
            jax.ShapeDtypeStruct((_NPROB * _PAD, _W), jnp.float32),
            jax.ShapeDtypeStruct((_NPROB * _PAD,), jnp.int32),
            jax.ShapeDtypeStruct((_NPROB * _NITEM,), jnp.int32),
        ],
        scratch_types=[
            pltpu.VMEM((_NITEM,), jnp.int32),
            pltpu.VMEM((_NG,), jnp.int32),
            pltpu.VMEM((_NG,), jnp.int32),
            pltpu.VMEM((_NITEM,), jnp.int32),
            pltpu.VMEM((_NITEM,), jnp.int32),
            pltpu.VMEM((_CHUNK,), jnp.int32),
            pltpu.VMEM((_CHUNK, _W), jnp.float32),
            pltpu.SemaphoreType.DMA,
        ],
    )
    return f(g_flat, hist_flat, qv_tab)


def _attn_body(bias_ref, a_ref, b_ref, ta_ref, tb_ref, x_ref):
    ab = jnp.concatenate([a_ref[0], b_ref[0]], axis=0)
    v_all = ab[:, :_D]
    k_all = ab[:, _D:]
    t_all = jnp.concatenate([ta_ref[0, 0, 0], tb_ref[0, 0, 0]], axis=0)
    q = k_all[32:32 + _QB] * (1.0 / (_D ** 0.5))
    qt = t_all[32:32 + _QB]

    dots = lax.dot_general(q, k_all, (((1,), (1,)), ((), ())),
                           preferred_element_type=jnp.float32)
    pen = bias_ref[...] - 100000.0 * (
        qt[:, None] == t_all[None, :]).astype(jnp.float32)
    e = jnp.exp(dots + pen)
    ssum = jnp.sum(e, axis=1, keepdims=True)
    x = lax.dot_general(e, v_all, (((1,), (0,)), ((), ())),
                        preferred_element_type=jnp.float32)
    x = x / ssum
    lse_b = jnp.broadcast_to(jnp.log(ssum), (_QB, _D))
    x_ref[0] = jnp.concatenate([x, lse_b], axis=1)


def _attn_tc(sqv_pad, tok_pad):
    tok_a = tok_pad.reshape(_NPROB, _PAD // _QB, 1, _QB)
    tok_b = tok_pad.reshape(_NPROB, _PAD // 64, 1, 64)
    nj = _NITEM // _QB
    r = lax.broadcasted_iota(jnp.int32, (_QB, _KB), 0)
    ccol = lax.broadcasted_iota(jnp.int32, (_QB, _KB), 1)
    lo = (r // _CW) * _CW
    band_bias = jnp.where((ccol >= lo) & (ccol < lo + 3 * _CW),
                          0.0, -1e30).astype(jnp.float32)
    return pl.pallas_call(
        _attn_body,
        grid=(_NPROB, nj),
        in_specs=[
            pl.BlockSpec((_QB, _KB), lambda p, j: (0, 0)),
            pl.BlockSpec((1, _QB, _W), lambda p, j: (p, j, 0)),
            pl.BlockSpec((1, 64, _W), lambda p, j: (p, 4 * j + 4, 0)),
            pl.BlockSpec((1, 1, 1, _QB), lambda p, j: (p, j, 0, 0)),
            pl.BlockSpec((1, 1, 1, 64), lambda p, j: (p, 4 * j + 4, 0, 0)),
        ],
        out_specs=pl.BlockSpec((1, _QB, _W), lambda p, j: (p, j, 0)),
        out_shape=jax.ShapeDtypeStruct((_NPROB, _NITEM, _W), jnp.float32),
    )(band_bias, sqv_pad, sqv_pad, tok_a, tok_b)


def _unsort_body(xtab_hbm, ipos_hbm, o_hbm, ipos_v, idx_v, rows_v, sem):
    p = lax.axis_index("s") * 2 + lax.axis_index("c")
    pltpu.sync_copy(ipos_hbm.at[pl.ds(p * _NITEM, _NITEM)], ipos_v)

    def chunk_step(c, _):
        base = c * _CHUNK

        def mk_idx(j, _):
            pos16 = ipos_v[pl.ds(base + j * _LANES, _LANES)]
            idx_v[pl.ds(j * _LANES, _LANES)] = p * _NITEM + pos16
            return 0

        lax.fori_loop(0, _CHUNK // _LANES, mk_idx, 0)
        pltpu.async_copy(xtab_hbm.at[idx_v], rows_v, sem).wait()
        pltpu.sync_copy(rows_v, o_hbm.at[pl.ds(p * _NITEM + base, _CHUNK)])
        return 0

    lax.fori_loop(0, _NITEM // _CHUNK, chunk_step, 0)


def _unsort_sc(xtab, ipos):
    mesh = plsc.VectorSubcoreMesh(core_axis_name="c", subcore_axis_name="s")
    f = pl.kernel(
        _unsort_body,
        mesh=mesh,
        compiler_params=pltpu.CompilerParams(needs_layout_passes=False),
        out_type=jax.ShapeDtypeStruct((_NPROB * _NITEM, _W), jnp.float32),
        scratch_types=[
            pltpu.VMEM((_NITEM,), jnp.int32),
            pltpu.VMEM((_CHUNK,), jnp.int32),
            pltpu.VMEM((_CHUNK, _W), jnp.float32),
            pltpu.SemaphoreType.DMA,
        ],
    )
    return f(xtab, ipos)


def _combine_body(o_ref, out_ref):
    blk = o_ref[0]
    o = blk[:, :, :_D]
    lgb = blk[:, :, _D:]
    m = jnp.max(lgb, axis=0, keepdims=True)
    e = jnp.exp(lgb - m)
    w = e / jnp.sum(e, axis=0, keepdims=True)
    out_ref[0] = jnp.sum(w * o, axis=0)


def _combine_tc(o_ext):
    o4 = o_ext.reshape(_NPROB, _NH, _S, _W)
    return pl.pallas_call(
        _combine_body,
        grid=(_NPROB, _S // 1024),
        in_specs=[
            pl.BlockSpec((1, _NH, 1024, _W), lambda p, t: (p, 0, t, 0)),
        ],
        out_specs=pl.BlockSpec((1, 1024, _D), lambda p, t: (p, t, 0)),
        out_shape=jax.ShapeDtypeStruct((_NPROB, _S, _D), jnp.float32),
    )(o4)


def kernel(query, key, value, rotations):
    del key
    B, S, H, D = query.shape
    q_p = query.transpose(0, 2, 1, 3).reshape(_NPROB, _S, _D)
    v_p = value.transpose(0, 2, 1, 3).reshape(_NPROB, _S, _D)
    qv_tab = jnp.concatenate([v_p, q_p], axis=-1).reshape(_NPROB * _S, _W)
    rot2d = rotations.reshape(_D, _NH * (_NB // 2))

    g, hist = _hash_tc(q_p, rot2d)
    sqv_flat, tok_flat, ipos = _sort_gather_sc(
        g.reshape(-1), hist.reshape(-1), qv_tab)
    x_ext = _attn_tc(sqv_flat.reshape(_NPROB, _PAD, _W),
                     tok_flat.reshape(_NPROB, _PAD))
    o_ext = _unsort_sc(x_ext.reshape(_NPROB * _NITEM, _W), ipos)
    out_p = _combine_tc(o_ext)
    return out_p.reshape(B, H, S, D).transpose(0, 2, 1, 3)


# speedup vs baseline: 1.1904x; 1.0359x over previous
import functools

import jax
import jax.numpy as jnp
from jax import lax
from jax.experimental import pallas as pl
from jax.experimental.pallas import tpu as pltpu
from jax.experimental.pallas import tpu_sc as plsc

_NB = 64
_NH = 4
_S = 2048
_D = 64
_NPROB = 32
_NITEM = _NH * _S
_NG = _NH * _NB
_CW = 32
_QB = 256
_KB = _QB + 64
_PAD = 33 * _QB
_CHUNK = 128
_LANES = 16
_W = 2 * _D


def _hash_body(q_ref, rot_ref, g_ref, hist_ref):
    q = q_ref[0]
    rot = rot_ref[...]
    r = lax.dot_general(q, rot, (((1,), (0,)), ((), ())),
                        preferred_element_type=jnp.float32)
    for h in range(_NH):
        rh = r[:, h * (_NB // 2):(h + 1) * (_NB // 2)]
        both = jnp.concatenate([rh, -rh], axis=1)
        bh = jnp.argmax(both, axis=1).astype(jnp.int32)
        g_ref[0, h, :] = bh + h * _NB
        onehot = (bh[:, None] ==
                  lax.broadcasted_iota(jnp.int32, (_S, _NB), 1))
        hist_ref[0, 0, pl.ds(h * _NB, _NB)] = jnp.sum(
            onehot.astype(jnp.int32), axis=0)


def _hash_tc(q_p, rot2d):
    return pl.pallas_call(
        _hash_body,
        grid=(_NPROB,),
        in_specs=[
            pl.BlockSpec((1, _S, _D), lambda p: (p, 0, 0)),
            pl.BlockSpec((_D, _NH * _NB // 2), lambda p: (0, 0)),
        ],
        out_specs=[
            pl.BlockSpec((1, _NH, _S), lambda p: (p, 0, 0)),
            pl.BlockSpec((1, 1, _NG), lambda p: (p, 0, 0)),
        ],
        out_shape=[
            jax.ShapeDtypeStruct((_NPROB, _NH, _S), jnp.int32),
            jax.ShapeDtypeStruct((_NPROB, 1, _NG), jnp.int32),
        ],
    )(q_p, rot2d)


def _sort_gather_body(g_hbm, hist_hbm, qv_hbm,
                      sqv_hbm, tok_hbm, ipos_hbm,
                      g_v, hist_v, start_v, ipos_v, st_v, idx_v,
                      rows_v, sem):
    p = lax.axis_index("s") * 2 + lax.axis_index("c")
    pltpu.sync_copy(g_hbm.at[pl.ds(p * _NITEM, _NITEM)], g_v)
    pltpu.sync_copy(hist_hbm.at[pl.ds(p * _NG, _NG)], hist_v)

    carry = jnp.int32(0)
    for bi in range(_NG // _LANES):
        h16 = hist_v[pl.ds(bi * _LANES, _LANES)]
        inc = plsc.cumsum(h16)
        start_v[pl.ds(bi * _LANES, _LANES)] = inc - h16 + carry
        carry = carry + inc[_LANES - 1]

    lane = lax.iota(jnp.int32, _LANES)

    def sort_step(i, _):
        g16 = g_v[pl.ds(i * _LANES, _LANES)]
        item = i * _LANES + lane
        skey, sitem = plsc.sort_key_val(g16 * _LANES + lane, item)
        gs = skey >> 4
        gprev = gs.at[jnp.maximum(lane - 1, 0)].get(mode="promise_in_bounds")
        run_start = (gs != gprev) | (lane == 0)
        rank = lane - plsc.cummax(jnp.where(run_start, lane, 0))
        c = plsc.load_gather(start_v, [gs])
        pos = c + rank
        plsc.store_scatter(ipos_v, [sitem], pos)
        plsc.store_scatter(st_v, [pos], sitem & (_S - 1))
        gnext = gs.at[jnp.minimum(lane + 1, _LANES - 1)].get(
            mode="promise_in_bounds")
        last = (gs != gnext) | (lane == _LANES - 1)
        plsc.store_scatter(start_v, [gs], pos + 1, mask=last)
        return 0

    lax.fori_loop(0, _NITEM // _LANES, sort_step, 0)

    pltpu.sync_copy(ipos_v, ipos_hbm.at[pl.ds(p * _NITEM, _NITEM)])

    def gather_rows(src_off, n, dst_off):
        def mk_idx(j, _):
            st16 = st_v[pl.ds(src_off + j * _LANES, _LANES)]
            idx_v[pl.ds(j * _LANES, _LANES)] = p * _S + st16
            return 0
        lax.fori_loop(0, n // _LANES, mk_idx, 0)
        pltpu.async_copy(qv_hbm.at[idx_v.at[pl.ds(0, n)]],
                         rows_v.at[pl.ds(0, n)], sem).wait()
        pltpu.sync_copy(rows_v.at[pl.ds(0, n)],
                        sqv_hbm.at[pl.ds(p * _PAD + dst_off, n)])
        pltpu.sync_copy(st_v.at[pl.ds(src_off, n)],
                        tok_hbm.at[pl.ds(p * _PAD + dst_off, n)])

    def chunk_step(c, _):
        gather_rows(c * _CHUNK, _CHUNK, 32 + c * _CHUNK)
        return 0

    lax.fori_loop(0, _NITEM // _CHUNK, chunk_step, 0)
    gather_rows(_NITEM - 32, 32, 0)
    gather_rows(0, 32, _NITEM + 32)


def _sort_gather_sc(g_flat, hist_flat, qv_tab):
    mesh = plsc.VectorSubcoreMesh(core_axis_name="c", subcore_axis_name="s")
    f = pl.kernel(
        _sort_gather_body,
        mesh=mesh,
        compiler_params=pltpu.CompilerParams(needs_layout_passes=False),
        out_type=[
            jax.ShapeDtypeStruct((_NPROB * _PAD, _W), jnp.float32),
            jax.ShapeDtypeStruct((_NPROB * _PAD,), jnp.int32),
            jax.ShapeDtypeStruct((_NPROB * _NITEM,), jnp.int32),
        ],
        scratch_types=[
            pltpu.VMEM((_NITEM,), jnp.int32),
            pltpu.VMEM((_NG,), jnp.int32),
            pltpu.VMEM((_NG,), jnp.int32),
            pltpu.VMEM((_NITEM,), jnp.int32),
            pltpu.VMEM((_NITEM,), jnp.int32),
            pltpu.VMEM((_CHUNK,), jnp.int32),
            pltpu.VMEM((_CHUNK, _W), jnp.float32),
            pltpu.SemaphoreType.DMA,
        ],
    )
    return f(g_flat, hist_flat, qv_tab)


def _attn_body(bias_ref, a_ref, b_ref, ta_ref, tb_ref, x_ref):
    ab = jnp.concatenate([a_ref[0], b_ref[0]], axis=0)
    k_all = ab[:, :_D]
    v_all = ab[:, _D:]
    t_all = jnp.concatenate([ta_ref[0, 0, 0], tb_ref[0, 0, 0]], axis=0)
    q = k_all[32:32 + _QB] * (1.0 / (_D ** 0.5))
    qt = t_all[32:32 + _QB]

    dots = lax.dot_general(q, k_all, (((1,), (1,)), ((), ())),
                           preferred_element_type=jnp.float32)
    pen = bias_ref[...] - 100000.0 * (
        qt[:, None] == t_all[None, :]).astype(jnp.float32)
    e = jnp.exp(dots + pen)
    ssum = jnp.sum(e, axis=1, keepdims=True)
    x = lax.dot_general(e, v_all, (((1,), (0,)), ((), ())),
                        preferred_element_type=jnp.float32)
    x = x / ssum
    lse_b = jnp.broadcast_to(jnp.log(ssum), (_QB, _D))
    x_ref[0] = jnp.concatenate([x, lse_b], axis=1)


def _attn_tc(sqv_pad, tok_pad):
    tok_a = tok_pad.reshape(_NPROB, _PAD // _QB, 1, _QB)
    tok_b = tok_pad.reshape(_NPROB, _PAD // 64, 1, 64)
    nj = _NITEM // _QB
    r = lax.broadcasted_iota(jnp.int32, (_QB, _KB), 0)
    ccol = lax.broadcasted_iota(jnp.int32, (_QB, _KB), 1)
    lo = (r // _CW) * _CW
    band_bias = jnp.where((ccol >= lo) & (ccol < lo + 3 * _CW),
                          0.0, -1e30).astype(jnp.float32)
    return pl.pallas_call(
        _attn_body,
        grid=(_NPROB, nj),
        in_specs=[
            pl.BlockSpec((_QB, _KB), lambda p, j: (0, 0)),
            pl.BlockSpec((1, _QB, _W), lambda p, j: (p, j, 0)),
            pl.BlockSpec((1, 64, _W), lambda p, j: (p, 4 * j + 4, 0)),
            pl.BlockSpec((1, 1, 1, _QB), lambda p, j: (p, j, 0, 0)),
            pl.BlockSpec((1, 1, 1, 64), lambda p, j: (p, 4 * j + 4, 0, 0)),
        ],
        out_specs=pl.BlockSpec((1, _QB, _W), lambda p, j: (p, j, 0)),
        out_shape=jax.ShapeDtypeStruct((_NPROB, _NITEM, _W), jnp.float32),
    )(band_bias, sqv_pad, sqv_pad, tok_a, tok_b)


def _unsort_body(xtab_hbm, ipos_hbm, o_hbm, ipos_v, idx_v, rows_v, sem):
    p = lax.axis_index("s") * 2 + lax.axis_index("c")
    pltpu.sync_copy(ipos_hbm.at[pl.ds(p * _NITEM, _NITEM)], ipos_v)

    def chunk_step(c, _):
        base = c * _CHUNK

        def mk_idx(j, _):
            pos16 = ipos_v[pl.ds(base + j * _LANES, _LANES)]
            idx_v[pl.ds(j * _LANES, _LANES)] = p * _NITEM + pos16
            return 0

        lax.fori_loop(0, _CHUNK // _LANES, mk_idx, 0)
        pltpu.async_copy(xtab_hbm.at[idx_v], rows_v, sem).wait()
        pltpu.sync_copy(rows_v, o_hbm.at[pl.ds(p * _NITEM + base, _CHUNK)])
        return 0

    lax.fori_loop(0, _NITEM // _CHUNK, chunk_step, 0)


def _unsort_sc(xtab, ipos):
    mesh = plsc.VectorSubcoreMesh(core_axis_name="c", subcore_axis_name="s")
    f = pl.kernel(
        _unsort_body,
        mesh=mesh,
        compiler_params=pltpu.CompilerParams(needs_layout_passes=False),
        out_type=jax.ShapeDtypeStruct((_NPROB * _NITEM, _W), jnp.float32),
        scratch_types=[
            pltpu.VMEM((_NITEM,), jnp.int32),
            pltpu.VMEM((_CHUNK,), jnp.int32),
            pltpu.VMEM((_CHUNK, _W), jnp.float32),
            pltpu.SemaphoreType.DMA,
        ],
    )
    return f(xtab, ipos)


def _combine_body(o_ref, out_ref):
    blk = o_ref[0]
    o = blk[:, :, :_D]
    lgb = blk[:, :, _D:]
    m = jnp.max(lgb, axis=0, keepdims=True)
    e = jnp.exp(lgb - m)
    w = e / jnp.sum(e, axis=0, keepdims=True)
    out_ref[0] = jnp.sum(w * o, axis=0)


def _combine_tc(o_ext):
    o4 = o_ext.reshape(_NPROB, _NH, _S, _W)
    return pl.pallas_call(
        _combine_body,
        grid=(_NPROB, _S // 1024),
        in_specs=[
            pl.BlockSpec((1, _NH, 1024, _W), lambda p, t: (p, 0, t, 0)),
        ],
        out_specs=pl.BlockSpec((1, 1024, _D), lambda p, t: (p, t, 0)),
        out_shape=jax.ShapeDtypeStruct((_NPROB, _S, _D), jnp.float32),
    )(o4)


def kernel(query, key, value, rotations):
    del key
    B, S, H, D = query.shape
    q_p = query.transpose(0, 2, 1, 3).reshape(_NPROB, _S, _D)
    v_p = value.transpose(0, 2, 1, 3).reshape(_NPROB, _S, _D)
    qv_tab = jnp.concatenate([q_p, v_p], axis=-1).reshape(_NPROB * _S, _W)
    rot2d = rotations.reshape(_D, _NH * (_NB // 2))

    g, hist = _hash_tc(q_p, rot2d)
    sqv_flat, tok_flat, ipos = _sort_gather_sc(
        g.reshape(-1), hist.reshape(-1), qv_tab)
    x_ext = _attn_tc(sqv_flat.reshape(_NPROB, _PAD, _W),
                     tok_flat.reshape(_NPROB, _PAD))
    o_ext = _unsort_sc(x_ext.reshape(_NPROB * _NITEM, _W), ipos)
    out_p = _combine_tc(o_ext)
    return out_p.reshape(B, H, S, D).transpose(0, 2, 1, 3)
